# baseline (speedup 1.0000x reference)
import math
import numpy as np
import jax
import jax.numpy as jnp
from jax.experimental import pallas as pl
from jax.experimental.pallas import tpu as pltpu

B = 16; T = 2048; DIM = 16; NP = 8; RPP = 16; OBS = 1.0; PRED = 96
LAT = 128; HEADS = 8; DFF = 256; LAYERS = 3
NQ = NP * RPP
HD = RPP // HEADS


def _pe_np(seq_len, d):
    pos = np.arange(seq_len, dtype=np.float32)[:, None]
    div = np.exp(np.arange(0, d, 2, dtype=np.float32) * -(math.log(10000.0) / d))
    pe = np.zeros((seq_len, d), dtype=np.float32)
    pe[:, 0::2] = np.sin(pos * div)
    pe[:, 1::2] = np.cos(pos * div)
    return pe


_REF_ROW = np.linspace(0.0, OBS, NQ).astype(np.float32)[None, :]
_EDGES = np.linspace(0.0, OBS, NP + 1).astype(np.float32)
_LO_ROW = _EDGES[np.arange(NQ) // RPP][None, :].astype(np.float32)
_HI_ROW = _EDGES[np.arange(NQ) // RPP + 1][None, :].astype(np.float32)
_PE_TILED = np.tile(_pe_np(NP, RPP), (DIM, 1))
_PAIR = np.zeros((RPP, HEADS), dtype=np.float32)
_PAIR[np.arange(RPP), np.arange(RPP) // HD] = 1.0


def _lnk(x, g, b):
    m = jnp.mean(x, axis=-1, keepdims=True)
    xc = x - m
    v = jnp.mean(xc * xc, axis=-1, keepdims=True)
    return xc * jax.lax.rsqrt(v + 1e-5) * g + b


def _body(data_ref, W1, b1, W2, b2, Wq, Wk, Wv, Wo, Wf1, bf1, Wf2, bf2,
          l1g, l1b, l2g, l2b, lfg, lfb, WlinR, blin,
          refr, lor, hir, peT, pair, out_ref):
    d = data_ref[0]
    vals = d[:, :DIM]
    msk = d[:, DIM:2 * DIM]
    time = d[:, 2 * DIM:2 * DIM + 1]

    v = vals * msk
    h1 = jnp.maximum(
        jnp.dot(v, W1[...], preferred_element_type=jnp.float32) + b1[...], 0.0)
    h = jnp.dot(h1, W2[...], preferred_element_type=jnp.float32) + b2[...]

    obs = jnp.sum(msk, axis=1, keepdims=True) > 0.0
    dt = time - refr[...]
    sc = -100.0 * dt * dt
    tm = (time >= lor[...]) & (time <= hir[...]) & obs
    sc = jnp.where(tm, sc, -1e9)
    mx = jnp.max(sc, axis=0, keepdims=True)
    e = jnp.exp(sc - mx)
    ssum = jnp.sum(e, axis=0, keepdims=True)
    hasf = jnp.max(jnp.where(tm, 1.0, 0.0), axis=0, keepdims=True)
    attnT = e * (hasf / ssum)
    repT = jnp.dot(h.T, attnT, preferred_element_type=jnp.float32)

    x3 = jnp.stack([repT[:, p * RPP:(p + 1) * RPP] for p in range(NP)], axis=1)
    x2 = x3.reshape(NQ, RPP) + peT[...]

    pm = pair[...]
    inv_sqrt_hd = 1.0 / math.sqrt(float(HD))
    for l in range(LAYERS):
        q = jnp.dot(x2, Wq[l], preferred_element_type=jnp.float32)
        k = jnp.dot(x2, Wk[l], preferred_element_type=jnp.float32)
        vv = jnp.dot(x2, Wv[l], preferred_element_type=jnp.float32)
        q3 = q.reshape(DIM, NP, RPP)
        k3 = k.reshape(DIM, NP, RPP)
        v3 = vv.reshape(DIM, NP, RPP)
        E = q3[:, :, None, :] * k3[:, None, :, :]
        S = jnp.dot(E.reshape(-1, RPP), pm,
                    preferred_element_type=jnp.float32) * inv_sqrt_hd
        S4 = S.reshape(DIM, NP, NP, HEADS)
        mS = jnp.max(S4, axis=2, keepdims=True)
        eS = jnp.exp(S4 - mS)
        A4 = eS / jnp.sum(eS, axis=2, keepdims=True)
        Afull = jnp.dot(A4.reshape(-1, HEADS), pm.T,
                        preferred_element_type=jnp.float32).reshape(DIM, NP, NP, RPP)
        O = jnp.sum(Afull * v3[:, None, :, :], axis=2)
        o = jnp.dot(O.reshape(NQ, RPP), Wo[l], preferred_element_type=jnp.float32)
        x2 = _lnk(x2 + o, l1g[l], l1b[l])
        y = jnp.dot(
            jax.nn.gelu(jnp.dot(x2, Wf1[l], preferred_element_type=jnp.float32) + bf1[l]),
            Wf2[l], preferred_element_type=jnp.float32) + bf2[l]
        x2 = _lnk(x2 + y, l2g[l], l2b[l])
    x2 = _lnk(x2, lfg[...], lfb[...])

    x3f = x2.reshape(DIM, NP, RPP)
    acc = jnp.dot(x3f[:, 0, :], WlinR[:, 0, :], preferred_element_type=jnp.float32)
    for p in range(1, NP):
        acc = acc + jnp.dot(x3f[:, p, :], WlinR[:, p, :],
                            preferred_element_type=jnp.float32)
    out_ref[0] = acc + blin[...]


def kernel(data, W1, b1, W2, b2, Wq, Wk, Wv, Wo, Wf1, bf1, Wf2, bf2,
           ln1g, ln1b, ln2g, ln2b, lnfg, lnfb, Wlin, blin):
    f32 = jnp.float32
    full = lambda shape: pl.BlockSpec(shape, lambda b: (0,) * len(shape))
    operands = [
        data,
        W1, b1.reshape(1, LAT), W2, b2.reshape(1, DIM),
        Wq, Wk, Wv, Wo,
        Wf1, bf1.reshape(LAYERS, 1, DFF), Wf2, bf2.reshape(LAYERS, 1, RPP),
        ln1g.reshape(LAYERS, 1, RPP), ln1b.reshape(LAYERS, 1, RPP),
        ln2g.reshape(LAYERS, 1, RPP), ln2b.reshape(LAYERS, 1, RPP),
        lnfg.reshape(1, RPP), lnfb.reshape(1, RPP),
        Wlin.reshape(RPP, NP, PRED), blin.reshape(1, PRED),
        jnp.asarray(_REF_ROW), jnp.asarray(_LO_ROW), jnp.asarray(_HI_ROW),
        jnp.asarray(_PE_TILED), jnp.asarray(_PAIR),
    ]
    in_specs = [pl.BlockSpec((1, T, 2 * DIM + 1), lambda b: (b, 0, 0))]
    in_specs += [full(op.shape) for op in operands[1:]]
    out = pl.pallas_call(
        _body,
        grid=(B,),
        in_specs=in_specs,
        out_specs=pl.BlockSpec((1, DIM, PRED), lambda b: (b, 0, 0)),
        out_shape=jax.ShapeDtypeStruct((B, DIM, PRED), f32),
        compiler_params=pltpu.CompilerParams(
            dimension_semantics=("parallel",)),
    )(*operands)
    return jnp.transpose(out, (0, 2, 1))


# speedup vs baseline: 1.0909x; 1.0909x over previous
import math
import numpy as np
import jax
import jax.numpy as jnp
from jax.experimental import pallas as pl
from jax.experimental.pallas import tpu as pltpu

B = 16; T = 2048; DIM = 16; NP = 8; RPP = 16; OBS = 1.0; PRED = 96
LAT = 128; HEADS = 8; DFF = 256; LAYERS = 3
NQ = NP * RPP
HD = RPP // HEADS
BPB = 2


def _pe_np(seq_len, d):
    pos = np.arange(seq_len, dtype=np.float32)[:, None]
    div = np.exp(np.arange(0, d, 2, dtype=np.float32) * -(math.log(10000.0) / d))
    pe = np.zeros((seq_len, d), dtype=np.float32)
    pe[:, 0::2] = np.sin(pos * div)
    pe[:, 1::2] = np.cos(pos * div)
    return pe


_REF_ROW = np.linspace(0.0, OBS, NQ).astype(np.float32)[None, :]
_EDGES = np.linspace(0.0, OBS, NP + 1).astype(np.float32)
_LO_ROW = _EDGES[np.arange(NQ) // RPP][None, :].astype(np.float32)
_HI_ROW = _EDGES[np.arange(NQ) // RPP + 1][None, :].astype(np.float32)
_PE_TILED = np.tile(_pe_np(NP, RPP), (DIM, 1))
_PAIR = np.zeros((RPP, HEADS), dtype=np.float32)
_PAIR[np.arange(RPP), np.arange(RPP) // HD] = 1.0


def _lnk(x, g, b):
    m = jnp.mean(x, axis=-1, keepdims=True)
    xc = x - m
    v = jnp.mean(xc * xc, axis=-1, keepdims=True)
    return xc * jax.lax.rsqrt(v + 1e-5) * g + b


def _one_batch(d, W1, b1, W2, b2, refr, lor, hir, peT):
    vals = d[:, :DIM]
    msk = d[:, DIM:2 * DIM]
    time = d[:, 2 * DIM:2 * DIM + 1]

    v = vals * msk
    h1 = jnp.maximum(
        jnp.dot(v, W1, preferred_element_type=jnp.float32) + b1, 0.0)
    h = jnp.dot(h1, W2, preferred_element_type=jnp.float32) + b2

    obs = jnp.sum(msk, axis=1, keepdims=True) > 0.0
    tf = jnp.where(obs, time, 2.0)
    dt = tf - refr
    tm = (tf >= lor) & (tf <= hir)
    e = jnp.where(tm, jnp.exp(-100.0 * dt * dt), 0.0)
    ssum = jnp.sum(e, axis=0, keepdims=True)
    scale = jnp.where(ssum > 0.0, 1.0 / ssum, 0.0)
    attnT = e * scale
    repT = jnp.dot(h.T, attnT, preferred_element_type=jnp.float32)

    x3 = jnp.stack([repT[:, p * RPP:(p + 1) * RPP] for p in range(NP)], axis=1)
    return x3.reshape(NQ, RPP) + peT


def _body(data_ref, W1, b1, W2, b2, Wq, Wk, Wv, Wo, Wf1, bf1, Wf2, bf2,
          l1g, l1b, l2g, l2b, lfg, lfb, WlinR, blin,
          refr, lor, hir, peT, pair, out_ref):
    xs = [_one_batch(data_ref[bb], W1[...], b1[...], W2[...], b2[...],
                     refr[...], lor[...], hir[...], peT[...])
          for bb in range(BPB)]
    x2 = jnp.concatenate(xs, axis=0)

    ND = BPB * DIM
    pm = pair[...]
    inv_sqrt_hd = 1.0 / math.sqrt(float(HD))
    for l in range(LAYERS):
        q = jnp.dot(x2, Wq[l], preferred_element_type=jnp.float32)
        k = jnp.dot(x2, Wk[l], preferred_element_type=jnp.float32)
        vv = jnp.dot(x2, Wv[l], preferred_element_type=jnp.float32)
        q3 = q.reshape(ND, NP, RPP)
        k3 = k.reshape(ND, NP, RPP)
        v3 = vv.reshape(ND, NP, RPP)
        E = q3[:, :, None, :] * k3[:, None, :, :]
        S = jnp.dot(E.reshape(-1, RPP), pm,
                    preferred_element_type=jnp.float32) * inv_sqrt_hd
        S4 = S.reshape(ND, NP, NP, HEADS)
        mS = jnp.max(S4, axis=2, keepdims=True)
        eS = jnp.exp(S4 - mS)
        A4 = eS / jnp.sum(eS, axis=2, keepdims=True)
        Afull = jnp.dot(A4.reshape(-1, HEADS), pm.T,
                        preferred_element_type=jnp.float32).reshape(ND, NP, NP, RPP)
        O = jnp.sum(Afull * v3[:, None, :, :], axis=2)
        o = jnp.dot(O.reshape(BPB * NQ, RPP), Wo[l], preferred_element_type=jnp.float32)
        x2 = _lnk(x2 + o, l1g[l], l1b[l])
        y = jnp.dot(
            jax.nn.gelu(jnp.dot(x2, Wf1[l], preferred_element_type=jnp.float32) + bf1[l]),
            Wf2[l], preferred_element_type=jnp.float32) + bf2[l]
        x2 = _lnk(x2 + y, l2g[l], l2b[l])
    x2 = _lnk(x2, lfg[...], lfb[...])

    x3f = x2.reshape(ND, NP, RPP)
    acc = jnp.dot(x3f[:, 0, :], WlinR[:, 0, :], preferred_element_type=jnp.float32)
    for p in range(1, NP):
        acc = acc + jnp.dot(x3f[:, p, :], WlinR[:, p, :],
                            preferred_element_type=jnp.float32)
    out_ref[...] = (acc + blin[...]).reshape(BPB, DIM, PRED)


def kernel(data, W1, b1, W2, b2, Wq, Wk, Wv, Wo, Wf1, bf1, Wf2, bf2,
           ln1g, ln1b, ln2g, ln2b, lnfg, lnfb, Wlin, blin):
    f32 = jnp.float32
    full = lambda shape: pl.BlockSpec(shape, lambda b: (0,) * len(shape))
    operands = [
        data,
        W1, b1.reshape(1, LAT), W2, b2.reshape(1, DIM),
        Wq, Wk, Wv, Wo,
        Wf1, bf1.reshape(LAYERS, 1, DFF), Wf2, bf2.reshape(LAYERS, 1, RPP),
        ln1g.reshape(LAYERS, 1, RPP), ln1b.reshape(LAYERS, 1, RPP),
        ln2g.reshape(LAYERS, 1, RPP), ln2b.reshape(LAYERS, 1, RPP),
        lnfg.reshape(1, RPP), lnfb.reshape(1, RPP),
        Wlin.reshape(RPP, NP, PRED), blin.reshape(1, PRED),
        jnp.asarray(_REF_ROW), jnp.asarray(_LO_ROW), jnp.asarray(_HI_ROW),
        jnp.asarray(_PE_TILED), jnp.asarray(_PAIR),
    ]
    in_specs = [pl.BlockSpec((BPB, T, 2 * DIM + 1), lambda b: (b, 0, 0))]
    in_specs += [full(op.shape) for op in operands[1:]]
    out = pl.pallas_call(
        _body,
        grid=(B // BPB,),
        in_specs=in_specs,
        out_specs=pl.BlockSpec((BPB, DIM, PRED), lambda b: (b, 0, 0)),
        out_shape=jax.ShapeDtypeStruct((B, DIM, PRED), f32),
        compiler_params=pltpu.CompilerParams(
            dimension_semantics=("parallel",)),
    )(*operands)
    return jnp.transpose(out, (0, 2, 1))


# speedup vs baseline: 1.1295x; 1.0354x over previous
import math
import numpy as np
import jax
import jax.numpy as jnp
from jax.experimental import pallas as pl
from jax.experimental.pallas import tpu as pltpu

B = 16; T = 2048; DIM = 16; NP = 8; RPP = 16; OBS = 1.0; PRED = 96
LAT = 128; HEADS = 8; DFF = 256; LAYERS = 3
NQ = NP * RPP
HD = RPP // HEADS
BPB = 2


def _pe_np(seq_len, d):
    pos = np.arange(seq_len, dtype=np.float32)[:, None]
    div = np.exp(np.arange(0, d, 2, dtype=np.float32) * -(math.log(10000.0) / d))
    pe = np.zeros((seq_len, d), dtype=np.float32)
    pe[:, 0::2] = np.sin(pos * div)
    pe[:, 1::2] = np.cos(pos * div)
    return pe


_REF_ROW = np.linspace(0.0, OBS, NQ).astype(np.float32)[None, :]
_EDGES = np.linspace(0.0, OBS, NP + 1).astype(np.float32)
_LO_ROW = _EDGES[np.arange(NQ) // RPP][None, :].astype(np.float32)
_HI_ROW = _EDGES[np.arange(NQ) // RPP + 1][None, :].astype(np.float32)
_PE_TILED = np.tile(_pe_np(NP, RPP), (DIM, 1))
_r = np.arange(RPP)
_TILEJ = np.zeros((RPP, NQ), dtype=np.float32)
for _j in range(NP):
    _TILEJ[_r, _j * RPP + _r] = 1.0
_PS2 = np.zeros((NQ, NP * HEADS), dtype=np.float32)
for _j in range(NP):
    _PS2[_j * RPP + _r, _j * HEADS + _r // HD] = 1.0
_MJSUM = np.zeros((NP * HEADS, NP * HEADS), dtype=np.float32)
for _j in range(NP):
    for _j2 in range(NP):
        for _h in range(HEADS):
            _MJSUM[_j * HEADS + _h, _j2 * HEADS + _h] = 1.0
_PSO = np.zeros((NP * HEADS, NQ), dtype=np.float32)
for _j in range(NP):
    _PSO[_j * HEADS + _r // HD, _j * RPP + _r] = 1.0
_SUMJ = np.zeros((NQ, RPP), dtype=np.float32)
for _j in range(NP):
    _SUMJ[_j * RPP + _r, _r] = 1.0
_M16 = np.full((RPP, RPP), 1.0 / RPP, dtype=np.float32)


def _lnk(x, g, b, m16):
    m = jnp.dot(x, m16, preferred_element_type=jnp.float32)
    xc = x - m
    v = jnp.dot(xc * xc, m16, preferred_element_type=jnp.float32)
    return xc * jax.lax.rsqrt(v + 1e-5) * g + b


def _one_batch(d, W1, b1, W2, b2, refr, lor, hir, peT):
    vals = d[:, :DIM]
    msk = d[:, DIM:2 * DIM]
    time = d[:, 2 * DIM:2 * DIM + 1]

    v = vals * msk
    h1 = jnp.maximum(
        jnp.dot(v, W1, preferred_element_type=jnp.float32) + b1, 0.0)
    h = jnp.dot(h1, W2, preferred_element_type=jnp.float32) + b2

    obs = jnp.sum(msk, axis=1, keepdims=True) > 0.0
    tf = jnp.where(obs, time, 2.0)
    dt = tf - refr
    tm = (tf >= lor) & (tf <= hir)
    e = jnp.where(tm, jnp.exp(-100.0 * dt * dt), 0.0)
    ssum = jnp.sum(e, axis=0, keepdims=True)
    scale = jnp.where(ssum > 0.0, 1.0 / ssum, 0.0)
    attnT = e * scale
    repT = jnp.dot(h.T, attnT, preferred_element_type=jnp.float32)

    x3 = jnp.stack([repT[:, p * RPP:(p + 1) * RPP] for p in range(NP)], axis=1)
    return x3.reshape(NQ, RPP) + peT


def _pack_b(a, nd):
    a3 = a.reshape(nd, NP, RPP)
    return jnp.concatenate([a3[:, p, :] for p in range(NP)], axis=1)


def _body(data_ref, W1, b1, W2, b2, Wq, Wk, Wv, Wo, Wf1, bf1, Wf2, bf2,
          l1g, l1b, l2g, l2b, lfg, lfb, WlinR, blin,
          refr, lor, hir, peT, tilej, ps2, mjsum, pso, sumj, m16, out_ref):
    xs = [_one_batch(data_ref[bb], W1[...], b1[...], W2[...], b2[...],
                     refr[...], lor[...], hir[...], peT[...])
          for bb in range(BPB)]
    x2 = jnp.concatenate(xs, axis=0)

    ND = BPB * DIM
    NR = BPB * NQ
    tj, p2, mj, po, sj, m16v = (tilej[...], ps2[...], mjsum[...], pso[...],
                                sumj[...], m16[...])
    inv_sqrt_hd = 1.0 / math.sqrt(float(HD))
    for l in range(LAYERS):
        q = jnp.dot(x2, Wq[l], preferred_element_type=jnp.float32)
        k = jnp.dot(x2, Wk[l], preferred_element_type=jnp.float32)
        vv = jnp.dot(x2, Wv[l], preferred_element_type=jnp.float32)
        kB = _pack_b(k, ND)
        vB = _pack_b(vv, ND)
        qE = jnp.dot(q, tj, preferred_element_type=jnp.float32)
        kE = jnp.broadcast_to(kB[:, None, :], (ND, NP, NQ)).reshape(NR, NQ)
        S = jnp.dot(qE * kE, p2,
                    preferred_element_type=jnp.float32) * inv_sqrt_hd
        mS = jnp.max(S, axis=1, keepdims=True)
        eS = jnp.exp(S - mS)
        Z = jnp.dot(eS, mj, preferred_element_type=jnp.float32)
        A = eS / Z
        AE = jnp.dot(A, po, preferred_element_type=jnp.float32)
        vE = jnp.broadcast_to(vB[:, None, :], (ND, NP, NQ)).reshape(NR, NQ)
        O = jnp.dot(AE * vE, sj, preferred_element_type=jnp.float32)
        o = jnp.dot(O, Wo[l], preferred_element_type=jnp.float32)
        x2 = _lnk(x2 + o, l1g[l], l1b[l], m16v)
        y = jnp.dot(
            jax.nn.gelu(jnp.dot(x2, Wf1[l], preferred_element_type=jnp.float32) + bf1[l]),
            Wf2[l], preferred_element_type=jnp.float32) + bf2[l]
        x2 = _lnk(x2 + y, l2g[l], l2b[l], m16v)
    x2 = _lnk(x2, lfg[...], lfb[...], m16v)

    x3f = x2.reshape(ND, NP, RPP)
    acc = jnp.dot(x3f[:, 0, :], WlinR[:, 0, :], preferred_element_type=jnp.float32)
    for p in range(1, NP):
        acc = acc + jnp.dot(x3f[:, p, :], WlinR[:, p, :],
                            preferred_element_type=jnp.float32)
    out_ref[...] = (acc + blin[...]).reshape(BPB, DIM, PRED)


def kernel(data, W1, b1, W2, b2, Wq, Wk, Wv, Wo, Wf1, bf1, Wf2, bf2,
           ln1g, ln1b, ln2g, ln2b, lnfg, lnfb, Wlin, blin):
    f32 = jnp.float32
    full = lambda shape: pl.BlockSpec(shape, lambda b: (0,) * len(shape))
    operands = [
        data,
        W1, b1.reshape(1, LAT), W2, b2.reshape(1, DIM),
        Wq, Wk, Wv, Wo,
        Wf1, bf1.reshape(LAYERS, 1, DFF), Wf2, bf2.reshape(LAYERS, 1, RPP),
        ln1g.reshape(LAYERS, 1, RPP), ln1b.reshape(LAYERS, 1, RPP),
        ln2g.reshape(LAYERS, 1, RPP), ln2b.reshape(LAYERS, 1, RPP),
        lnfg.reshape(1, RPP), lnfb.reshape(1, RPP),
        Wlin.reshape(RPP, NP, PRED), blin.reshape(1, PRED),
        jnp.asarray(_REF_ROW), jnp.asarray(_LO_ROW), jnp.asarray(_HI_ROW),
        jnp.asarray(_PE_TILED), jnp.asarray(_TILEJ), jnp.asarray(_PS2),
        jnp.asarray(_MJSUM), jnp.asarray(_PSO), jnp.asarray(_SUMJ),
        jnp.asarray(_M16),
    ]
    in_specs = [pl.BlockSpec((BPB, T, 2 * DIM + 1), lambda b: (b, 0, 0))]
    in_specs += [full(op.shape) for op in operands[1:]]
    out = pl.pallas_call(
        _body,
        grid=(B // BPB,),
        in_specs=in_specs,
        out_specs=pl.BlockSpec((BPB, DIM, PRED), lambda b: (b, 0, 0)),
        out_shape=jax.ShapeDtypeStruct((B, DIM, PRED), f32),
        compiler_params=pltpu.CompilerParams(
            dimension_semantics=("parallel",)),
    )(*operands)
    return jnp.transpose(out, (0, 2, 1))


# speedup vs baseline: 1.2639x; 1.1190x over previous
import math
import numpy as np
import jax
import jax.numpy as jnp
from jax.experimental import pallas as pl
from jax.experimental.pallas import tpu as pltpu

B = 16; T = 2048; DIM = 16; NP = 8; RPP = 16; OBS = 1.0; PRED = 96
LAT = 128; HEADS = 8; DFF = 256; LAYERS = 3
NQ = NP * RPP
HD = RPP // HEADS
BPB = 4


def _pe_np(seq_len, d):
    pos = np.arange(seq_len, dtype=np.float32)[:, None]
    div = np.exp(np.arange(0, d, 2, dtype=np.float32) * -(math.log(10000.0) / d))
    pe = np.zeros((seq_len, d), dtype=np.float32)
    pe[:, 0::2] = np.sin(pos * div)
    pe[:, 1::2] = np.cos(pos * div)
    return pe


_REF_ROW = np.linspace(0.0, OBS, NQ).astype(np.float32)[None, :]
_EDGES = np.linspace(0.0, OBS, NP + 1).astype(np.float32)
_LO_ROW = _EDGES[np.arange(NQ) // RPP][None, :].astype(np.float32)
_HI_ROW = _EDGES[np.arange(NQ) // RPP + 1][None, :].astype(np.float32)
_PE_TILED = np.tile(_pe_np(NP, RPP), (DIM, 1))
_r = np.arange(RPP)
_TILEJ = np.zeros((RPP, NQ), dtype=np.float32)
for _j in range(NP):
    _TILEJ[_r, _j * RPP + _r] = 1.0
_PS2 = np.zeros((NQ, NP * HEADS), dtype=np.float32)
for _j in range(NP):
    _PS2[_j * RPP + _r, _j * HEADS + _r // HD] = 1.0
_MJSUM = np.zeros((NP * HEADS, NP * HEADS), dtype=np.float32)
for _j in range(NP):
    for _j2 in range(NP):
        for _h in range(HEADS):
            _MJSUM[_j * HEADS + _h, _j2 * HEADS + _h] = 1.0
_PSO = np.zeros((NP * HEADS, NQ), dtype=np.float32)
for _j in range(NP):
    _PSO[_j * HEADS + _r // HD, _j * RPP + _r] = 1.0
_SUMJ = np.zeros((NQ, RPP), dtype=np.float32)
for _j in range(NP):
    _SUMJ[_j * RPP + _r, _r] = 1.0
_M16 = np.full((RPP, RPP), 1.0 / RPP, dtype=np.float32)


def _lnk(x, g, b, m16):
    m = jnp.dot(x, m16, preferred_element_type=jnp.float32)
    xc = x - m
    v = jnp.dot(xc * xc, m16, preferred_element_type=jnp.float32)
    return xc * jax.lax.rsqrt(v + 1e-5) * g + b


def _one_batch(d, W1, b1, W2, b2, refr, lor, hir, peT):
    vals = d[:, :DIM]
    msk = d[:, DIM:2 * DIM]
    time = d[:, 2 * DIM:2 * DIM + 1]

    v = vals * msk
    h1 = jnp.maximum(
        jnp.dot(v, W1, preferred_element_type=jnp.float32) + b1, 0.0)
    h = jnp.dot(h1, W2, preferred_element_type=jnp.float32) + b2

    obs = jnp.sum(msk, axis=1, keepdims=True) > 0.0
    tf = jnp.where(obs, time, 2.0)
    dt = tf - refr
    tm = (tf >= lor) & (tf <= hir)
    e = jnp.where(tm, jnp.exp(-100.0 * dt * dt), 0.0)
    ssum = jnp.sum(e, axis=0, keepdims=True)
    scale = jnp.where(ssum > 0.0, 1.0 / ssum, 0.0)
    attnT = e * scale
    repT = jnp.dot(h.T, attnT, preferred_element_type=jnp.float32)

    x3 = jnp.stack([repT[:, p * RPP:(p + 1) * RPP] for p in range(NP)], axis=1)
    return x3.reshape(NQ, RPP) + peT


def _pack_b(a, nd):
    a3 = a.reshape(nd, NP, RPP)
    return jnp.concatenate([a3[:, p, :] for p in range(NP)], axis=1)


def _body(data_ref, W1, b1, W2, b2, Wq, Wk, Wv, Wo, Wf1, bf1, Wf2, bf2,
          l1g, l1b, l2g, l2b, lfg, lfb, WlinR, blin,
          refr, lor, hir, peT, tilej, ps2, mjsum, pso, sumj, m16, out_ref):
    xs = [_one_batch(data_ref[bb], W1[...], b1[...], W2[...], b2[...],
                     refr[...], lor[...], hir[...], peT[...])
          for bb in range(BPB)]
    x2 = jnp.concatenate(xs, axis=0)

    ND = BPB * DIM
    NR = BPB * NQ
    tj, p2, mj, po, sj, m16v = (tilej[...], ps2[...], mjsum[...], pso[...],
                                sumj[...], m16[...])
    inv_sqrt_hd = 1.0 / math.sqrt(float(HD))
    for l in range(LAYERS):
        q = jnp.dot(x2, Wq[l], preferred_element_type=jnp.float32)
        k = jnp.dot(x2, Wk[l], preferred_element_type=jnp.float32)
        vv = jnp.dot(x2, Wv[l], preferred_element_type=jnp.float32)
        kB = _pack_b(k, ND)
        vB = _pack_b(vv, ND)
        qE = jnp.dot(q, tj, preferred_element_type=jnp.float32)
        kE = jnp.broadcast_to(kB[:, None, :], (ND, NP, NQ)).reshape(NR, NQ)
        S = jnp.dot(qE * kE, p2,
                    preferred_element_type=jnp.float32) * inv_sqrt_hd
        mS = jnp.max(S, axis=1, keepdims=True)
        eS = jnp.exp(S - mS)
        Z = jnp.dot(eS, mj, preferred_element_type=jnp.float32)
        A = eS / Z
        AE = jnp.dot(A, po, preferred_element_type=jnp.float32)
        vE = jnp.broadcast_to(vB[:, None, :], (ND, NP, NQ)).reshape(NR, NQ)
        O = jnp.dot(AE * vE, sj, preferred_element_type=jnp.float32)
        o = jnp.dot(O, Wo[l], preferred_element_type=jnp.float32)
        x2 = _lnk(x2 + o, l1g[l], l1b[l], m16v)
        y = jnp.dot(
            jax.nn.gelu(jnp.dot(x2, Wf1[l], preferred_element_type=jnp.float32) + bf1[l]),
            Wf2[l], preferred_element_type=jnp.float32) + bf2[l]
        x2 = _lnk(x2 + y, l2g[l], l2b[l], m16v)
    x2 = _lnk(x2, lfg[...], lfb[...], m16v)

    x3f = x2.reshape(ND, NP, RPP)
    acc = jnp.dot(x3f[:, 0, :], WlinR[:, 0, :], preferred_element_type=jnp.float32)
    for p in range(1, NP):
        acc = acc + jnp.dot(x3f[:, p, :], WlinR[:, p, :],
                            preferred_element_type=jnp.float32)
    out_ref[...] = (acc + blin[...]).reshape(BPB, DIM, PRED)


def kernel(data, W1, b1, W2, b2, Wq, Wk, Wv, Wo, Wf1, bf1, Wf2, bf2,
           ln1g, ln1b, ln2g, ln2b, lnfg, lnfb, Wlin, blin):
    f32 = jnp.float32
    full = lambda shape: pl.BlockSpec(shape, lambda b: (0,) * len(shape))
    operands = [
        data,
        W1, b1.reshape(1, LAT), W2, b2.reshape(1, DIM),
        Wq, Wk, Wv, Wo,
        Wf1, bf1.reshape(LAYERS, 1, DFF), Wf2, bf2.reshape(LAYERS, 1, RPP),
        ln1g.reshape(LAYERS, 1, RPP), ln1b.reshape(LAYERS, 1, RPP),
        ln2g.reshape(LAYERS, 1, RPP), ln2b.reshape(LAYERS, 1, RPP),
        lnfg.reshape(1, RPP), lnfb.reshape(1, RPP),
        Wlin.reshape(RPP, NP, PRED), blin.reshape(1, PRED),
        jnp.asarray(_REF_ROW), jnp.asarray(_LO_ROW), jnp.asarray(_HI_ROW),
        jnp.asarray(_PE_TILED), jnp.asarray(_TILEJ), jnp.asarray(_PS2),
        jnp.asarray(_MJSUM), jnp.asarray(_PSO), jnp.asarray(_SUMJ),
        jnp.asarray(_M16),
    ]
    in_specs = [pl.BlockSpec((BPB, T, 2 * DIM + 1), lambda b: (b, 0, 0))]
    in_specs += [full(op.shape) for op in operands[1:]]
    out = pl.pallas_call(
        _body,
        grid=(B // BPB,),
        in_specs=in_specs,
        out_specs=pl.BlockSpec((BPB, DIM, PRED), lambda b: (b, 0, 0)),
        out_shape=jax.ShapeDtypeStruct((B, DIM, PRED), f32),
        compiler_params=pltpu.CompilerParams(
            dimension_semantics=("parallel",)),
    )(*operands)
    return jnp.transpose(out, (0, 2, 1))


# speedup vs baseline: 1.5622x; 1.2360x over previous
import math
import numpy as np
import jax
import jax.numpy as jnp
from jax.experimental import pallas as pl
from jax.experimental.pallas import tpu as pltpu

B = 16; T = 2048; DIM = 16; NP = 8; RPP = 16; OBS = 1.0; PRED = 96
LAT = 128; HEADS = 8; DFF = 256; LAYERS = 3
NQ = NP * RPP
HD = RPP // HEADS
BPB = 8


def _pe_np(seq_len, d):
    pos = np.arange(seq_len, dtype=np.float32)[:, None]
    div = np.exp(np.arange(0, d, 2, dtype=np.float32) * -(math.log(10000.0) / d))
    pe = np.zeros((seq_len, d), dtype=np.float32)
    pe[:, 0::2] = np.sin(pos * div)
    pe[:, 1::2] = np.cos(pos * div)
    return pe


_REF_ROW = np.linspace(0.0, OBS, NQ).astype(np.float32)[None, :]
_EDGES = np.linspace(0.0, OBS, NP + 1).astype(np.float32)
_LO_ROW = _EDGES[np.arange(NQ) // RPP][None, :].astype(np.float32)
_HI_ROW = _EDGES[np.arange(NQ) // RPP + 1][None, :].astype(np.float32)
_PE_TILED = np.tile(_pe_np(NP, RPP), (DIM, 1))
_r = np.arange(RPP)
_TILEJ = np.zeros((RPP, NQ), dtype=np.float32)
for _j in range(NP):
    _TILEJ[_r, _j * RPP + _r] = 1.0
_PS2 = np.zeros((NQ, NP * HEADS), dtype=np.float32)
for _j in range(NP):
    _PS2[_j * RPP + _r, _j * HEADS + _r // HD] = 1.0
_MJSUM = np.zeros((NP * HEADS, NP * HEADS), dtype=np.float32)
for _j in range(NP):
    for _j2 in range(NP):
        for _h in range(HEADS):
            _MJSUM[_j * HEADS + _h, _j2 * HEADS + _h] = 1.0
_PSO = np.zeros((NP * HEADS, NQ), dtype=np.float32)
for _j in range(NP):
    _PSO[_j * HEADS + _r // HD, _j * RPP + _r] = 1.0
_SUMJ = np.zeros((NQ, RPP), dtype=np.float32)
for _j in range(NP):
    _SUMJ[_j * RPP + _r, _r] = 1.0
_M16 = np.full((RPP, RPP), 1.0 / RPP, dtype=np.float32)


def _lnk(x, g, b, m16):
    m = jnp.dot(x, m16, preferred_element_type=jnp.float32)
    xc = x - m
    v = jnp.dot(xc * xc, m16, preferred_element_type=jnp.float32)
    return xc * jax.lax.rsqrt(v + 1e-5) * g + b


def _one_batch(d, W1, b1, W2, b2, refr, lor, hir, peT):
    vals = d[:, :DIM]
    msk = d[:, DIM:2 * DIM]
    time = d[:, 2 * DIM:2 * DIM + 1]

    v = vals * msk
    h1 = jnp.maximum(
        jnp.dot(v, W1, preferred_element_type=jnp.float32) + b1, 0.0)
    h = jnp.dot(h1, W2, preferred_element_type=jnp.float32) + b2

    obs = jnp.sum(msk, axis=1, keepdims=True) > 0.0
    tf = jnp.where(obs, time, 2.0)
    dt = tf - refr
    tm = (tf >= lor) & (tf <= hir)
    e = jnp.where(tm, jnp.exp(-100.0 * dt * dt), 0.0)
    ssum = jnp.sum(e, axis=0, keepdims=True)
    scale = jnp.where(ssum > 0.0, 1.0 / ssum, 0.0)
    attnT = e * scale
    repT = jnp.dot(h.T, attnT, preferred_element_type=jnp.float32)

    x3 = jnp.stack([repT[:, p * RPP:(p + 1) * RPP] for p in range(NP)], axis=1)
    return x3.reshape(NQ, RPP) + peT


def _pack_b(a, nd):
    a3 = a.reshape(nd, NP, RPP)
    return jnp.concatenate([a3[:, p, :] for p in range(NP)], axis=1)


def _body(data_ref, W1, b1, W2, b2, Wq, Wk, Wv, Wo, Wf1, bf1, Wf2, bf2,
          l1g, l1b, l2g, l2b, lfg, lfb, WlinR, blin,
          refr, lor, hir, peT, tilej, ps2, mjsum, pso, sumj, m16, out_ref):
    xs = [_one_batch(data_ref[bb], W1[...], b1[...], W2[...], b2[...],
                     refr[...], lor[...], hir[...], peT[...])
          for bb in range(BPB)]
    x2 = jnp.concatenate(xs, axis=0)

    ND = BPB * DIM
    NR = BPB * NQ
    tj, p2, mj, po, sj, m16v = (tilej[...], ps2[...], mjsum[...], pso[...],
                                sumj[...], m16[...])
    inv_sqrt_hd = 1.0 / math.sqrt(float(HD))
    for l in range(LAYERS):
        q = jnp.dot(x2, Wq[l], preferred_element_type=jnp.float32)
        k = jnp.dot(x2, Wk[l], preferred_element_type=jnp.float32)
        vv = jnp.dot(x2, Wv[l], preferred_element_type=jnp.float32)
        kB = _pack_b(k, ND)
        vB = _pack_b(vv, ND)
        qE = jnp.dot(q, tj, preferred_element_type=jnp.float32)
        kE = jnp.broadcast_to(kB[:, None, :], (ND, NP, NQ)).reshape(NR, NQ)
        S = jnp.dot(qE * kE, p2,
                    preferred_element_type=jnp.float32) * inv_sqrt_hd
        mS = jnp.max(S, axis=1, keepdims=True)
        eS = jnp.exp(S - mS)
        Z = jnp.dot(eS, mj, preferred_element_type=jnp.float32)
        A = eS / Z
        AE = jnp.dot(A, po, preferred_element_type=jnp.float32)
        vE = jnp.broadcast_to(vB[:, None, :], (ND, NP, NQ)).reshape(NR, NQ)
        O = jnp.dot(AE * vE, sj, preferred_element_type=jnp.float32)
        o = jnp.dot(O, Wo[l], preferred_element_type=jnp.float32)
        x2 = _lnk(x2 + o, l1g[l], l1b[l], m16v)
        y = jnp.dot(
            jax.nn.gelu(jnp.dot(x2, Wf1[l], preferred_element_type=jnp.float32) + bf1[l]),
            Wf2[l], preferred_element_type=jnp.float32) + bf2[l]
        x2 = _lnk(x2 + y, l2g[l], l2b[l], m16v)
    x2 = _lnk(x2, lfg[...], lfb[...], m16v)

    x3f = x2.reshape(ND, NP, RPP)
    acc = jnp.dot(x3f[:, 0, :], WlinR[:, 0, :], preferred_element_type=jnp.float32)
    for p in range(1, NP):
        acc = acc + jnp.dot(x3f[:, p, :], WlinR[:, p, :],
                            preferred_element_type=jnp.float32)
    out_ref[...] = (acc + blin[...]).reshape(BPB, DIM, PRED)


def kernel(data, W1, b1, W2, b2, Wq, Wk, Wv, Wo, Wf1, bf1, Wf2, bf2,
           ln1g, ln1b, ln2g, ln2b, lnfg, lnfb, Wlin, blin):
    f32 = jnp.float32
    full = lambda shape: pl.BlockSpec(shape, lambda b: (0,) * len(shape))
    operands = [
        data,
        W1, b1.reshape(1, LAT), W2, b2.reshape(1, DIM),
        Wq, Wk, Wv, Wo,
        Wf1, bf1.reshape(LAYERS, 1, DFF), Wf2, bf2.reshape(LAYERS, 1, RPP),
        ln1g.reshape(LAYERS, 1, RPP), ln1b.reshape(LAYERS, 1, RPP),
        ln2g.reshape(LAYERS, 1, RPP), ln2b.reshape(LAYERS, 1, RPP),
        lnfg.reshape(1, RPP), lnfb.reshape(1, RPP),
        Wlin.reshape(RPP, NP, PRED), blin.reshape(1, PRED),
        jnp.asarray(_REF_ROW), jnp.asarray(_LO_ROW), jnp.asarray(_HI_ROW),
        jnp.asarray(_PE_TILED), jnp.asarray(_TILEJ), jnp.asarray(_PS2),
        jnp.asarray(_MJSUM), jnp.asarray(_PSO), jnp.asarray(_SUMJ),
        jnp.asarray(_M16),
    ]
    in_specs = [pl.BlockSpec((BPB, T, 2 * DIM + 1), lambda b: (b, 0, 0))]
    in_specs += [full(op.shape) for op in operands[1:]]
    out = pl.pallas_call(
        _body,
        grid=(B // BPB,),
        in_specs=in_specs,
        out_specs=pl.BlockSpec((BPB, DIM, PRED), lambda b: (b, 0, 0)),
        out_shape=jax.ShapeDtypeStruct((B, DIM, PRED), f32),
        compiler_params=pltpu.CompilerParams(
            dimension_semantics=("parallel",)),
    )(*operands)
    return jnp.transpose(out, (0, 2, 1))


# speedup vs baseline: 1.5702x; 1.0051x over previous
import math
import numpy as np
import jax
import jax.numpy as jnp
from jax.experimental import pallas as pl
from jax.experimental.pallas import tpu as pltpu

B = 16; T = 2048; DIM = 16; NP = 8; RPP = 16; OBS = 1.0; PRED = 96
LAT = 128; HEADS = 8; DFF = 256; LAYERS = 3
NQ = NP * RPP
HD = RPP // HEADS
BPB = 16


def _pe_np(seq_len, d):
    pos = np.arange(seq_len, dtype=np.float32)[:, None]
    div = np.exp(np.arange(0, d, 2, dtype=np.float32) * -(math.log(10000.0) / d))
    pe = np.zeros((seq_len, d), dtype=np.float32)
    pe[:, 0::2] = np.sin(pos * div)
    pe[:, 1::2] = np.cos(pos * div)
    return pe


_REF_ROW = np.linspace(0.0, OBS, NQ).astype(np.float32)[None, :]
_EDGES = np.linspace(0.0, OBS, NP + 1).astype(np.float32)
_LO_ROW = _EDGES[np.arange(NQ) // RPP][None, :].astype(np.float32)
_HI_ROW = _EDGES[np.arange(NQ) // RPP + 1][None, :].astype(np.float32)
_PE_TILED = np.tile(_pe_np(NP, RPP), (DIM, 1))
_r = np.arange(RPP)
_TILEJ = np.zeros((RPP, NQ), dtype=np.float32)
for _j in range(NP):
    _TILEJ[_r, _j * RPP + _r] = 1.0
_PS2 = np.zeros((NQ, NP * HEADS), dtype=np.float32)
for _j in range(NP):
    _PS2[_j * RPP + _r, _j * HEADS + _r // HD] = 1.0
_MJSUM = np.zeros((NP * HEADS, NP * HEADS), dtype=np.float32)
for _j in range(NP):
    for _j2 in range(NP):
        for _h in range(HEADS):
            _MJSUM[_j * HEADS + _h, _j2 * HEADS + _h] = 1.0
_PSO = np.zeros((NP * HEADS, NQ), dtype=np.float32)
for _j in range(NP):
    _PSO[_j * HEADS + _r // HD, _j * RPP + _r] = 1.0
_SUMJ = np.zeros((NQ, RPP), dtype=np.float32)
for _j in range(NP):
    _SUMJ[_j * RPP + _r, _r] = 1.0
_M16 = np.full((RPP, RPP), 1.0 / RPP, dtype=np.float32)


def _lnk(x, g, b, m16):
    m = jnp.dot(x, m16, preferred_element_type=jnp.float32)
    xc = x - m
    v = jnp.dot(xc * xc, m16, preferred_element_type=jnp.float32)
    return xc * jax.lax.rsqrt(v + 1e-5) * g + b


def _one_batch(d, W1, b1, W2, b2, refr, lor, hir, peT):
    vals = d[:, :DIM]
    msk = d[:, DIM:2 * DIM]
    time = d[:, 2 * DIM:2 * DIM + 1]

    v = vals * msk
    h1 = jnp.maximum(
        jnp.dot(v, W1, preferred_element_type=jnp.float32) + b1, 0.0)
    h = jnp.dot(h1, W2, preferred_element_type=jnp.float32) + b2

    obs = jnp.sum(msk, axis=1, keepdims=True) > 0.0
    tf = jnp.where(obs, time, 2.0)
    dt = tf - refr
    tm = (tf >= lor) & (tf <= hir)
    e = jnp.where(tm, jnp.exp(-100.0 * dt * dt), 0.0)
    ssum = jnp.sum(e, axis=0, keepdims=True)
    scale = jnp.where(ssum > 0.0, 1.0 / ssum, 0.0)
    attnT = e * scale
    repT = jnp.dot(h.T, attnT, preferred_element_type=jnp.float32)

    x3 = jnp.stack([repT[:, p * RPP:(p + 1) * RPP] for p in range(NP)], axis=1)
    return x3.reshape(NQ, RPP) + peT


def _pack_b(a, nd):
    a3 = a.reshape(nd, NP, RPP)
    return jnp.concatenate([a3[:, p, :] for p in range(NP)], axis=1)


def _body(data_ref, W1, b1, W2, b2, Wq, Wk, Wv, Wo, Wf1, bf1, Wf2, bf2,
          l1g, l1b, l2g, l2b, lfg, lfb, WlinR, blin,
          refr, lor, hir, peT, tilej, ps2, mjsum, pso, sumj, m16, out_ref):
    xs = [_one_batch(data_ref[bb], W1[...], b1[...], W2[...], b2[...],
                     refr[...], lor[...], hir[...], peT[...])
          for bb in range(BPB)]
    x2 = jnp.concatenate(xs, axis=0)

    ND = BPB * DIM
    NR = BPB * NQ
    tj, p2, mj, po, sj, m16v = (tilej[...], ps2[...], mjsum[...], pso[...],
                                sumj[...], m16[...])
    inv_sqrt_hd = 1.0 / math.sqrt(float(HD))
    for l in range(LAYERS):
        q = jnp.dot(x2, Wq[l], preferred_element_type=jnp.float32)
        k = jnp.dot(x2, Wk[l], preferred_element_type=jnp.float32)
        vv = jnp.dot(x2, Wv[l], preferred_element_type=jnp.float32)
        kB = _pack_b(k, ND)
        vB = _pack_b(vv, ND)
        qE = jnp.dot(q, tj, preferred_element_type=jnp.float32)
        kE = jnp.broadcast_to(kB[:, None, :], (ND, NP, NQ)).reshape(NR, NQ)
        S = jnp.dot(qE * kE, p2,
                    preferred_element_type=jnp.float32) * inv_sqrt_hd
        mS = jnp.max(S, axis=1, keepdims=True)
        eS = jnp.exp(S - mS)
        Z = jnp.dot(eS, mj, preferred_element_type=jnp.float32)
        A = eS / Z
        AE = jnp.dot(A, po, preferred_element_type=jnp.float32)
        vE = jnp.broadcast_to(vB[:, None, :], (ND, NP, NQ)).reshape(NR, NQ)
        O = jnp.dot(AE * vE, sj, preferred_element_type=jnp.float32)
        o = jnp.dot(O, Wo[l], preferred_element_type=jnp.float32)
        x2 = _lnk(x2 + o, l1g[l], l1b[l], m16v)
        y = jnp.dot(
            jax.nn.gelu(jnp.dot(x2, Wf1[l], preferred_element_type=jnp.float32) + bf1[l]),
            Wf2[l], preferred_element_type=jnp.float32) + bf2[l]
        x2 = _lnk(x2 + y, l2g[l], l2b[l], m16v)
    x2 = _lnk(x2, lfg[...], lfb[...], m16v)

    x3f = x2.reshape(ND, NP, RPP)
    acc = jnp.dot(x3f[:, 0, :], WlinR[:, 0, :], preferred_element_type=jnp.float32)
    for p in range(1, NP):
        acc = acc + jnp.dot(x3f[:, p, :], WlinR[:, p, :],
                            preferred_element_type=jnp.float32)
    out_ref[...] = (acc + blin[...]).reshape(BPB, DIM, PRED)


def kernel(data, W1, b1, W2, b2, Wq, Wk, Wv, Wo, Wf1, bf1, Wf2, bf2,
           ln1g, ln1b, ln2g, ln2b, lnfg, lnfb, Wlin, blin):
    f32 = jnp.float32
    full = lambda shape: pl.BlockSpec(shape, lambda b: (0,) * len(shape))
    operands = [
        data,
        W1, b1.reshape(1, LAT), W2, b2.reshape(1, DIM),
        Wq, Wk, Wv, Wo,
        Wf1, bf1.reshape(LAYERS, 1, DFF), Wf2, bf2.reshape(LAYERS, 1, RPP),
        ln1g.reshape(LAYERS, 1, RPP), ln1b.reshape(LAYERS, 1, RPP),
        ln2g.reshape(LAYERS, 1, RPP), ln2b.reshape(LAYERS, 1, RPP),
        lnfg.reshape(1, RPP), lnfb.reshape(1, RPP),
        Wlin.reshape(RPP, NP, PRED), blin.reshape(1, PRED),
        jnp.asarray(_REF_ROW), jnp.asarray(_LO_ROW), jnp.asarray(_HI_ROW),
        jnp.asarray(_PE_TILED), jnp.asarray(_TILEJ), jnp.asarray(_PS2),
        jnp.asarray(_MJSUM), jnp.asarray(_PSO), jnp.asarray(_SUMJ),
        jnp.asarray(_M16),
    ]
    in_specs = [pl.BlockSpec((BPB, T, 2 * DIM + 1), lambda b: (b, 0, 0))]
    in_specs += [full(op.shape) for op in operands[1:]]
    out = pl.pallas_call(
        _body,
        grid=(B // BPB,),
        in_specs=in_specs,
        out_specs=pl.BlockSpec((BPB, DIM, PRED), lambda b: (b, 0, 0)),
        out_shape=jax.ShapeDtypeStruct((B, DIM, PRED), f32),
        compiler_params=pltpu.CompilerParams(
            dimension_semantics=("parallel",)),
    )(*operands)
    return jnp.transpose(out, (0, 2, 1))


# speedup vs baseline: 1.6895x; 1.0760x over previous
import math
import numpy as np
import jax
import jax.numpy as jnp
from jax.experimental import pallas as pl
from jax.experimental.pallas import tpu as pltpu

B = 16; T = 2048; DIM = 16; NP = 8; RPP = 16; OBS = 1.0; PRED = 96
LAT = 128; HEADS = 8; DFF = 256; LAYERS = 3
NQ = NP * RPP
HD = RPP // HEADS
BPB = 16


def _pe_np(seq_len, d):
    pos = np.arange(seq_len, dtype=np.float32)[:, None]
    div = np.exp(np.arange(0, d, 2, dtype=np.float32) * -(math.log(10000.0) / d))
    pe = np.zeros((seq_len, d), dtype=np.float32)
    pe[:, 0::2] = np.sin(pos * div)
    pe[:, 1::2] = np.cos(pos * div)
    return pe


_REF_ROW = np.linspace(0.0, OBS, NQ).astype(np.float32)[None, :]
_EDGES = np.linspace(0.0, OBS, NP + 1).astype(np.float32)
_LO_ROW = _EDGES[np.arange(NQ) // RPP][None, :].astype(np.float32)
_HI_ROW = _EDGES[np.arange(NQ) // RPP + 1][None, :].astype(np.float32)
_PE_TILED = np.tile(_pe_np(NP, RPP), (DIM, 1))
_r = np.arange(RPP)
_TILEJ = np.zeros((RPP, NQ), dtype=np.float32)
for _j in range(NP):
    _TILEJ[_r, _j * RPP + _r] = 1.0
_PS2 = np.zeros((NQ, NP * HEADS), dtype=np.float32)
for _j in range(NP):
    _PS2[_j * RPP + _r, _j * HEADS + _r // HD] = 1.0
_MJSUM = np.zeros((NP * HEADS, NP * HEADS), dtype=np.float32)
for _j in range(NP):
    for _j2 in range(NP):
        for _h in range(HEADS):
            _MJSUM[_j * HEADS + _h, _j2 * HEADS + _h] = 1.0
_PSO = np.zeros((NP * HEADS, NQ), dtype=np.float32)
for _j in range(NP):
    _PSO[_j * HEADS + _r // HD, _j * RPP + _r] = 1.0
_SUMJ = np.zeros((NQ, RPP), dtype=np.float32)
for _j in range(NP):
    _SUMJ[_j * RPP + _r, _r] = 1.0
_M16 = np.full((RPP, RPP), 1.0 / RPP, dtype=np.float32)


def _lnk(x, g, b, m16):
    m = jnp.dot(x, m16, preferred_element_type=jnp.float32)
    xc = x - m
    v = jnp.dot(xc * xc, m16, preferred_element_type=jnp.float32)
    return xc * jax.lax.rsqrt(v + 1e-5) * g + b


def _one_batch(d, W1, b1, W2, b2, refr, lor, hir, peT, m16v):
    vals = d[:, :DIM]
    msk = d[:, DIM:2 * DIM]
    time = d[:, 2 * DIM:2 * DIM + 1]

    v = vals * msk
    h1 = jnp.maximum(
        jnp.dot(v, W1, preferred_element_type=jnp.float32) + b1, 0.0)
    h = jnp.dot(h1, W2, preferred_element_type=jnp.float32) + b2

    obs = jnp.dot(msk, m16v, preferred_element_type=jnp.float32)[:, :1] > 0.0
    tf = jnp.where(obs, time, 2.0)
    tfb = jnp.broadcast_to(tf, (T, NQ))
    dt = tfb - refr
    tm = (tfb >= lor) & (tfb <= hir)
    e = jnp.where(tm, jnp.exp(-100.0 * dt * dt), 0.0)
    ssum = jnp.sum(e, axis=0, keepdims=True)
    scale = jnp.where(ssum > 0.0, 1.0 / ssum, 0.0)
    attnT = e * scale
    repT = jnp.dot(h.T, attnT, preferred_element_type=jnp.float32)

    x3 = jnp.stack([repT[:, p * RPP:(p + 1) * RPP] for p in range(NP)], axis=1)
    return x3.reshape(NQ, RPP) + peT


def _pack_b(a, nd):
    a3 = a.reshape(nd, NP, RPP)
    return jnp.concatenate([a3[:, p, :] for p in range(NP)], axis=1)


def _body(data_ref, W1, b1, W2, b2, Wqkv, Wo, Wf1, bf1, Wf2, bf2,
          l1g, l1b, l2g, l2b, lfg, lfb, WlinP, blin,
          refr, lor, hir, peT, tilej, ps2, mjsum, pso, sumj, m16, out_ref):
    tj, p2, mj, po, sj, m16v = (tilej[...], ps2[...], mjsum[...], pso[...],
                                sumj[...], m16[...])
    W1v, b1v, W2v, b2v = W1[...], b1[...], W2[...], b2[...]
    refv, lov, hiv, pev = refr[...], lor[...], hir[...], peT[...]
    xs = [_one_batch(data_ref[bb], W1v, b1v, W2v, b2v, refv, lov, hiv, pev,
                     m16v)
          for bb in range(BPB)]
    x2 = jnp.concatenate(xs, axis=0)

    ND = BPB * DIM
    NR = BPB * NQ
    inv_sqrt_hd = 1.0 / math.sqrt(float(HD))
    for l in range(LAYERS):
        qkv = jnp.dot(x2, Wqkv[l], preferred_element_type=jnp.float32)
        q = qkv[:, :RPP]
        k = qkv[:, RPP:2 * RPP]
        vv = qkv[:, 2 * RPP:3 * RPP]
        kB = _pack_b(k, ND)
        vB = _pack_b(vv, ND)
        qE = jnp.dot(q, tj, preferred_element_type=jnp.float32)
        kE = jnp.broadcast_to(kB[:, None, :], (ND, NP, NQ)).reshape(NR, NQ)
        S = jnp.dot(qE * kE, p2,
                    preferred_element_type=jnp.float32) * inv_sqrt_hd
        mS = jnp.max(S, axis=1, keepdims=True)
        eS = jnp.exp(S - mS)
        Z = jnp.dot(eS, mj, preferred_element_type=jnp.float32)
        A = eS / Z
        AE = jnp.dot(A, po, preferred_element_type=jnp.float32)
        vE = jnp.broadcast_to(vB[:, None, :], (ND, NP, NQ)).reshape(NR, NQ)
        O = jnp.dot(AE * vE, sj, preferred_element_type=jnp.float32)
        o = jnp.dot(O, Wo[l], preferred_element_type=jnp.float32)
        x2 = _lnk(x2 + o, l1g[l], l1b[l], m16v)
        y = jnp.dot(
            jax.nn.gelu(jnp.dot(x2, Wf1[l], preferred_element_type=jnp.float32) + bf1[l]),
            Wf2[l], preferred_element_type=jnp.float32) + bf2[l]
        x2 = _lnk(x2 + y, l2g[l], l2b[l], m16v)
    x2 = _lnk(x2, lfg[...], lfb[...], m16v)

    xB = _pack_b(x2, ND)
    acc = jnp.dot(xB, WlinP[...], preferred_element_type=jnp.float32)
    out_ref[...] = (acc + blin[...]).reshape(BPB, DIM, PRED)


def kernel(data, W1, b1, W2, b2, Wq, Wk, Wv, Wo, Wf1, bf1, Wf2, bf2,
           ln1g, ln1b, ln2g, ln2b, lnfg, lnfb, Wlin, blin):
    f32 = jnp.float32
    full = lambda shape: pl.BlockSpec(shape, lambda b: (0,) * len(shape))
    operands = [
        data,
        W1, b1.reshape(1, LAT), W2, b2.reshape(1, DIM),
        jnp.concatenate([Wq, Wk, Wv], axis=2), Wo,
        Wf1, bf1.reshape(LAYERS, 1, DFF), Wf2, bf2.reshape(LAYERS, 1, RPP),
        ln1g.reshape(LAYERS, 1, RPP), ln1b.reshape(LAYERS, 1, RPP),
        ln2g.reshape(LAYERS, 1, RPP), ln2b.reshape(LAYERS, 1, RPP),
        lnfg.reshape(1, RPP), lnfb.reshape(1, RPP),
        Wlin.reshape(RPP, NP, PRED).transpose(1, 0, 2).reshape(NQ, PRED),
        blin.reshape(1, PRED),
        jnp.asarray(_REF_ROW), jnp.asarray(_LO_ROW), jnp.asarray(_HI_ROW),
        jnp.asarray(_PE_TILED), jnp.asarray(_TILEJ), jnp.asarray(_PS2),
        jnp.asarray(_MJSUM), jnp.asarray(_PSO), jnp.asarray(_SUMJ),
        jnp.asarray(_M16),
    ]
    in_specs = [pl.BlockSpec((BPB, T, 2 * DIM + 1), lambda b: (b, 0, 0))]
    in_specs += [full(op.shape) for op in operands[1:]]
    out = pl.pallas_call(
        _body,
        grid=(B // BPB,),
        in_specs=in_specs,
        out_specs=pl.BlockSpec((BPB, DIM, PRED), lambda b: (b, 0, 0)),
        out_shape=jax.ShapeDtypeStruct((B, DIM, PRED), f32),
        compiler_params=pltpu.CompilerParams(
            dimension_semantics=("parallel",)),
    )(*operands)
    return jnp.transpose(out, (0, 2, 1))
